# Initial kernel scaffold; baseline (speedup 1.0000x reference)
import jax
import jax.numpy as jnp
from jax.experimental import pallas as pl


def kernel(x, pe_table):
    raise NotImplementedError("write your pallas kernel here")


# speedup vs baseline: 2.1987x; 2.1987x over previous
import jax
import jax.numpy as jnp
from jax.experimental import pallas as pl


S_BLK = 512


def _add_kernel(x_ref, pe_ref, o_ref):
    o_ref[...] = x_ref[...] + pe_ref[...]


def kernel(x, pe_table):
    B, S, F = x.shape
    grid = (S // S_BLK,)
    return pl.pallas_call(
        _add_kernel,
        grid=grid,
        in_specs=[
            pl.BlockSpec((B, S_BLK, F), lambda i: (0, i, 0)),
            pl.BlockSpec((S_BLK, F), lambda i: (i, 0)),
        ],
        out_specs=pl.BlockSpec((B, S_BLK, F), lambda i: (0, i, 0)),
        out_shape=jax.ShapeDtypeStruct((B, S, F), x.dtype),
    )(x, pe_table)
